# baseline (speedup 1.0000x reference)
import functools

import jax
import jax.numpy as jnp
from jax import lax
from jax.experimental import pallas as pl
from jax.experimental.pallas import tpu as pltpu

_N = 16384
_C = 1000
_K = _N - int(0.75 * _N)
_BLK = 2048
_GRID = _N // _BLK


def _select_loss(g):
    b = lax.bitcast_convert_type(g, jnp.uint32)
    neg = (b >> jnp.uint32(31)) == jnp.uint32(1)
    key = jnp.where(neg, ~b, b | jnp.uint32(0x80000000))

    def body(i, prefix):
        t = prefix | (jnp.uint32(1) << (jnp.uint32(31) - i.astype(jnp.uint32)))
        cnt = jnp.sum((key < t).astype(jnp.int32))
        return jnp.where(cnt <= _K - 1, t, prefix)

    v = lax.fori_loop(0, 32, body, jnp.uint32(0))

    lt = key < v
    cnt_lt = jnp.sum(lt.astype(jnp.int32))
    sum_lt = jnp.sum(jnp.where(lt, g, jnp.float32(0.0)))
    vb = jnp.where((v >> jnp.uint32(31)) == jnp.uint32(1),
                   v ^ jnp.uint32(0x80000000), ~v)
    gv = lax.bitcast_convert_type(vb, jnp.float32)
    total = sum_lt + (jnp.float32(_K) - cnt_lt.astype(jnp.float32)) * gv
    return -total / jnp.float32(_K)


def _fused_body(y_hat_ref, y_ref, out_ref, g_ref):
    step = pl.program_id(0)
    y = y_ref[...]
    yh = y_hat_ref[...]
    col = lax.broadcasted_iota(jnp.int32, (_BLK, _C), 1)
    idx = jnp.argmax(y, axis=1, keepdims=True).astype(jnp.int32)
    g2 = jnp.where(col == idx, yh, jnp.float32(0.0))
    g = jnp.sum(g2, axis=1)
    g_ref[pl.ds(step * _BLK, _BLK)] = g

    @pl.when(step == _GRID - 1)
    def _():
        out_ref[0, 0] = _select_loss(g_ref[...])


def kernel(y_hat, y):
    out = pl.pallas_call(
        _fused_body,
        grid=(_GRID,),
        in_specs=[
            pl.BlockSpec((_BLK, _C), lambda i: (i, 0)),
            pl.BlockSpec((_BLK, _C), lambda i: (i, 0)),
        ],
        out_specs=pl.BlockSpec(memory_space=pltpu.SMEM),
        out_shape=jax.ShapeDtypeStruct((1, 1), jnp.float32),
        scratch_shapes=[pltpu.VMEM((_N,), jnp.float32)],
        compiler_params=pltpu.CompilerParams(
            dimension_semantics=("arbitrary",),
        ),
    )(y_hat, y)
    return out[0, 0]


# speedup vs baseline: 1.0013x; 1.0013x over previous
import functools

import jax
import jax.numpy as jnp
from jax import lax
from jax.experimental import pallas as pl
from jax.experimental.pallas import tpu as pltpu

_N = 16384
_C = 1000
_K = _N - int(0.75 * _N)
_BLK = 2048
_GRID = _N // _BLK


def _select_loss(g):
    b = lax.bitcast_convert_type(g, jnp.uint32)
    neg = (b >> jnp.uint32(31)) == jnp.uint32(1)
    key = jnp.where(neg, ~b, b | jnp.uint32(0x80000000))

    def body(i, prefix):
        t = prefix | (jnp.uint32(1) << (jnp.uint32(31) - i.astype(jnp.uint32)))
        cnt = jnp.sum((key < t).astype(jnp.int32))
        return jnp.where(cnt <= _K - 1, t, prefix)

    v = lax.fori_loop(0, 32, body, jnp.uint32(0))

    lt = key < v
    cnt_lt = jnp.sum(lt.astype(jnp.int32))
    sum_lt = jnp.sum(jnp.where(lt, g, jnp.float32(0.0)))
    vb = jnp.where((v >> jnp.uint32(31)) == jnp.uint32(1),
                   v ^ jnp.uint32(0x80000000), ~v)
    gv = lax.bitcast_convert_type(vb, jnp.float32)
    total = sum_lt + (jnp.float32(_K) - cnt_lt.astype(jnp.float32)) * gv
    return -total / jnp.float32(_K)


def _fused_body(y_hat_ref, y_ref, out_ref, g_ref):
    step = pl.program_id(0)
    y = y_ref[...]
    yh = y_hat_ref[...]
    m = jnp.max(y, axis=1, keepdims=True)
    col = lax.broadcasted_iota(jnp.int32, (_BLK, _C), 1)
    idx = jnp.min(jnp.where(y == m, col, jnp.int32(_C)), axis=1, keepdims=True)
    g2 = jnp.where(col == idx, yh, jnp.float32(0.0))
    g = jnp.sum(g2, axis=1)
    g_ref[pl.ds(step * _BLK, _BLK)] = g

    @pl.when(step == _GRID - 1)
    def _():
        out_ref[0, 0] = _select_loss(g_ref[...])


def kernel(y_hat, y):
    out = pl.pallas_call(
        _fused_body,
        grid=(_GRID,),
        in_specs=[
            pl.BlockSpec((_BLK, _C), lambda i: (i, 0)),
            pl.BlockSpec((_BLK, _C), lambda i: (i, 0)),
        ],
        out_specs=pl.BlockSpec(memory_space=pltpu.SMEM),
        out_shape=jax.ShapeDtypeStruct((1, 1), jnp.float32),
        scratch_shapes=[pltpu.VMEM((_N,), jnp.float32)],
        compiler_params=pltpu.CompilerParams(
            dimension_semantics=("arbitrary",),
        ),
    )(y_hat, y)
    return out[0, 0]
